# baseline (speedup 1.0000x reference)
import functools

import jax
import jax.numpy as jnp
from jax import lax
from jax.experimental import pallas as pl
from jax.experimental.pallas import tpu as pltpu
from jax.experimental.pallas import tpu_sc as plsc

B = 4096
NSAMP = 128
R = B * NSAMP
X = 4096
D = 64
L = 16

NC = 2
NSC = 16
NW = NC * NSC
BW = B // NW
NB = 2
CH = NB * NSAMP
NCHUNK = BW // NB

_ROT_A = (13, 15, 26, 6)
_ROT_B = (17, 29, 16, 24)
_PARITY = 0x1BD11BDA


def _rotl(x, r):
    return (x << r) | lax.shift_right_logical(x, 32 - r)


def _threefry_index(k0, k1, ks2, x1init):
    ks = (k0, k1, ks2)
    x0 = k0
    x1 = x1init + k1
    for g in range(5):
        rots = _ROT_A if g % 2 == 0 else _ROT_B
        for r in rots:
            x0 = x0 + x1
            x1 = _rotl(x1, r)
            x1 = x1 ^ x0
        x0 = x0 + ks[(g + 1) % 3]
        x1 = x1 + ks[(g + 2) % 3] + (g + 1)
    return (x0 ^ x1) & (X - 1)


_mesh = plsc.VectorSubcoreMesh(core_axis_name="c", subcore_axis_name="s")


@functools.partial(
    pl.kernel,
    out_type=[
        jax.ShapeDtypeStruct((R,), jnp.int32),
        jax.ShapeDtypeStruct((B, D, NSAMP), jnp.float32),
    ],
    mesh=_mesh,
    compiler_params=pltpu.CompilerParams(needs_layout_passes=False,
                                         use_tc_tiling_on_sc=False),
    scratch_types=[
        pltpu.VMEM((2, L), jnp.int32),
        pltpu.VMEM((X,), jnp.int32),
        pltpu.VMEM((2, NB, NSAMP), jnp.int32),
        pltpu.VMEM((BW * NSAMP,), jnp.int32),
        pltpu.VMEM((2, NB, NSAMP, D), jnp.float32),
        pltpu.VMEM((2, NB, D, NSAMP), jnp.float32),
        pltpu.SemaphoreType.DMA,
        pltpu.SemaphoreType.DMA,
        pltpu.SemaphoreType.DMA,
        pltpu.SemaphoreType.DMA,
    ],
)
def _sampler(keys_hbm, ids_hbm, emb_hbm, ids_out, emb_out,
             keys_v, tab_v, idx_v, oid_v, rows_v, trans_v,
             gsem0, gsem1, osem0, osem1):
    wid = lax.axis_index("s") * NC + lax.axis_index("c")
    b_base = wid * BW
    pltpu.sync_copy(keys_hbm, keys_v)
    pltpu.sync_copy(ids_hbm, tab_v)
    k0 = keys_v[0, :]
    k1 = keys_v[1, :]
    ks2 = k0 ^ k1 ^ _PARITY
    lane = lax.iota(jnp.int32, L)
    gsems = (gsem0, gsem1)
    osems = (osem0, osem1)
    zero = jnp.full((L,), 0, jnp.int32)

    def compute_idx2(c, p):
        row0 = (b_base + c * NB) * NSAMP
        loc0 = c * CH
        for q in range(NB):
            def vreg_body(j, cc, q=q):
                x1init = lane + (row0 + q * NSAMP + j * L)
                idx = _threefry_index(k0, k1, ks2, x1init)
                idx_v[p, q, pl.ds(j * L, L)] = idx
                oid_v[pl.ds(loc0 + q * NSAMP + j * L, L)] = plsc.load_gather(
                    tab_v, [idx])
                return cc
            lax.fori_loop(0, NSAMP // L, vreg_body, 0)

    def fire_gather(c, p):
        del c
        return [
            pltpu.async_copy(emb_hbm.at[idx_v.at[p, q]], rows_v.at[p, q],
                             gsems[p])
            for q in range(NB)
        ]

    def wait_gather(p):
        for q in range(NB):
            pltpu.make_async_copy(emb_hbm.at[idx_v.at[p, q]],
                                  rows_v.at[p, q], gsems[p]).wait()

    lane64 = lane * D

    def transpose(p):
        zp = zero + p

        def k_body(k, cc):
            c = (lane + k) & (D - 1)
            rbase = lane64 + c
            wbase = (c << 7) + lane
            for q in range(NB):
                for t in range(NSAMP // L):
                    roff = (q * NSAMP + t * L) * D
                    woff = q * (D * NSAMP) + t * L
                    v = plsc.load_gather(
                        rows_v, [zp, zero, zero, rbase + roff])
                    plsc.store_scatter(
                        trans_v, [zp, zero, zero, wbase + woff], v)
            return cc
        lax.fori_loop(0, D, k_body, 0)

    def fire_out(c, p):
        b0 = b_base + c * NB
        return pltpu.async_copy(trans_v.at[p], emb_out.at[pl.ds(b0, NB)],
                                osems[p])

    def wait_out(c, p):
        b0 = b_base + c * NB
        pltpu.make_async_copy(trans_v.at[p], emb_out.at[pl.ds(b0, NB)],
                              osems[p]).wait()

    compute_idx2(0, 0)
    fire_gather(0, 0)
    compute_idx2(1, 1)
    fire_gather(1, 1)

    def body(gg, carry):
        for p in range(2):
            g = 2 * gg + p
            wait_gather(p)

            @pl.when(gg >= 1)
            def _():
                wait_out(g - 2, p)

            transpose(p)
            fire_out(g, p)

            @pl.when(g + 2 < NCHUNK)
            def _():
                compute_idx2(g + 2, p)
                fire_gather(g + 2, p)
        return carry

    lax.fori_loop(0, NCHUNK // 2, body, 0)
    wait_out(NCHUNK - 2, 0)
    wait_out(NCHUNK - 1, 1)
    pltpu.sync_copy(oid_v, ids_out.at[pl.ds(b_base * NSAMP, BW * NSAMP)])


def kernel(positive_ids, num_to_sample, sampled_candidate_ids,
           sampled_candidate_embeddings):
    del positive_ids, num_to_sample
    kd = jax.random.key_data(jax.random.split(jax.random.key(42))[1])
    keys = lax.bitcast_convert_type(kd, jnp.int32)
    keys2d = jnp.broadcast_to(keys[:, None], (2, L))
    ids_flat, emb_bdn = _sampler(
        keys2d, sampled_candidate_ids, sampled_candidate_embeddings)
    return (ids_flat.reshape(B, NSAMP),
            jnp.transpose(emb_bdn, (0, 2, 1)))
